# baseline (speedup 1.0000x reference)
import functools

import jax
import jax.numpy as jnp
from jax import lax
from jax.experimental import pallas as pl
from jax.experimental.pallas import tpu as pltpu
from jax.experimental.pallas import tpu_sc as plsc

_N = 10000
_E = 320000
_DIN = 128
_HID = 64
_NH = 4
_OUT = 32
_G = 128

_NS = 16
_NP = 10240
_RPT = _NP // _NS
_CW = 128
_CN = 128
_NCHN = _E // _CN
_NCHW = _E // 2 // _CW

_mesh = plsc.VectorSubcoreMesh(core_axis_name="c", subcore_axis_name="s")
_SC_PARAMS = pltpu.CompilerParams(use_tc_tiling_on_sc=False)

_GDN = lax.GatherDimensionNumbers(
    offset_dims=(), collapsed_slice_dims=(0,), start_index_map=(0,))


def _bcast(x16, idx16):
    return lax.gather(x16, idx16[:, None], _GDN, (1,),
                      mode=lax.GatherScatterMode.PROMISE_IN_BOUNDS)


@functools.partial(
    pl.kernel,
    out_type=jax.ShapeDtypeStruct((2, _NP, 8), jnp.float32),
    mesh=_mesh,
    compiler_params=_SC_PARAMS,
    scratch_types=[
        pltpu.VMEM((_CW,), jnp.int32),
        pltpu.VMEM((_CW, 8), jnp.float32),
        pltpu.VMEM_SHARED((_NP, 8), jnp.float32),
    ],
)
def _p0_deg(dst_hbm, ones_hbm, zero8_hbm, out_hbm, didx, ones_v, acc):
    c = lax.axis_index("c")
    s = lax.axis_index("s")
    r0 = s * _RPT
    pltpu.sync_copy(zero8_hbm.at[pl.ds(r0, _RPT)], acc.at[pl.ds(r0, _RPT)])
    pltpu.sync_copy(ones_hbm, ones_v)
    plsc.subcore_barrier()
    nc = jnp.where(s < _NCHW - _NS * (_NCHW // _NS), _NCHW // _NS + 1,
                   _NCHW // _NS)

    def body(j, carry):
        base = (c * _NCHW + s + _NS * j) * _CW
        pltpu.sync_copy(dst_hbm.at[pl.ds(base, _CW)], didx)
        pltpu.sync_copy(ones_v, acc.at[didx], add=True)
        return carry

    lax.fori_loop(0, nc, body, 0)
    plsc.subcore_barrier()
    pltpu.sync_copy(acc.at[pl.ds(r0, _RPT)], out_hbm.at[c, pl.ds(r0, _RPT)])


@functools.partial(
    pl.kernel,
    out_type=jax.ShapeDtypeStruct((2, _NP, _HID), jnp.float32),
    mesh=_mesh,
    compiler_params=_SC_PARAMS,
    scratch_types=[
        pltpu.VMEM((_CW,), jnp.int32),
        pltpu.VMEM((_CW,), jnp.int32),
        pltpu.VMEM((_CW, _HID), jnp.float32),
        pltpu.VMEM_SHARED((_NP, _HID), jnp.float32),
    ],
)
def _pgcn(tab_hbm, src_hbm, dst_hbm, zero_hbm, out_hbm, sidx, didx, rows, acc):
    c = lax.axis_index("c")
    s = lax.axis_index("s")
    r0 = s * _RPT
    pltpu.sync_copy(zero_hbm.at[pl.ds(r0, _RPT)], acc.at[pl.ds(r0, _RPT)])
    plsc.subcore_barrier()
    nc = jnp.where(s < _NCHW - _NS * (_NCHW // _NS), _NCHW // _NS + 1,
                   _NCHW // _NS)

    def body(j, carry):
        base = (c * _NCHW + s + _NS * j) * _CW
        pltpu.sync_copy(src_hbm.at[pl.ds(base, _CW)], sidx)
        pltpu.sync_copy(dst_hbm.at[pl.ds(base, _CW)], didx)
        pltpu.sync_copy(tab_hbm.at[sidx], rows)
        pltpu.sync_copy(rows, acc.at[didx], add=True)
        return carry

    lax.fori_loop(0, nc, body, 0)
    plsc.subcore_barrier()
    pltpu.sync_copy(acc.at[pl.ds(r0, _RPT)], out_hbm.at[c, pl.ds(r0, _RPT)])


@functools.partial(
    pl.kernel,
    out_type=[
        jax.ShapeDtypeStruct((2, _E, 16), jnp.float32),
        jax.ShapeDtypeStruct((2, 2, _NP, _HID), jnp.float32),
        jax.ShapeDtypeStruct((2, _NP, 16), jnp.float32),
    ],
    mesh=_mesh,
    compiler_params=_SC_PARAMS,
    scratch_types=[
        pltpu.VMEM((_CN,), jnp.int32),
        pltpu.VMEM((_CN,), jnp.int32),
        pltpu.VMEM((_CN, _HID), jnp.float32),
        pltpu.VMEM((_CN, 16), jnp.float32),
        pltpu.VMEM((_CN, 16), jnp.float32),
        pltpu.VMEM((_CN, 16), jnp.float32),
        pltpu.VMEM((_CN, _HID), jnp.float32),
        pltpu.VMEM((_CN, _HID), jnp.float32),
        pltpu.VMEM_SHARED((_NP, _HID), jnp.float32),
        pltpu.VMEM_SHARED((_NP, _HID), jnp.float32),
        pltpu.VMEM_SHARED((_NP, 16), jnp.float32),
    ],
)
def _p3_gat(h_hbm, asrc_hbm, adst_hbm, src_hbm, dst_hbm, zero_hbm, zero16_hbm,
            ex_out, acc_out, den_out,
            sidx, didx, hrows, arows, brows, exv, scA, scB, accA, accB, den):
    c = lax.axis_index("c")
    s = lax.axis_index("s")
    r0 = s * _RPT
    pltpu.sync_copy(zero_hbm.at[pl.ds(r0, _RPT)], accA.at[pl.ds(r0, _RPT)])
    pltpu.sync_copy(zero_hbm.at[pl.ds(r0, _RPT)], accB.at[pl.ds(r0, _RPT)])
    pltpu.sync_copy(zero16_hbm.at[pl.ds(r0, _RPT)], den.at[pl.ds(r0, _RPT)])
    plsc.subcore_barrier()
    nc = jnp.where(s < _NCHN - _NS * (_NCHN // _NS), _NCHN // _NS + 1,
                   _NCHN // _NS)
    lane0 = jnp.zeros((16,), jnp.int32)
    lane1 = jnp.ones((16,), jnp.int32)

    def body(j, carry):
        chunk = s + _NS * j
        base = chunk * _CN
        pltpu.sync_copy(src_hbm.at[pl.ds(base, _CN)], sidx)
        pltpu.sync_copy(dst_hbm.at[pl.ds(base, _CN)], didx)
        pltpu.sync_copy(h_hbm.at[sidx], hrows)
        pltpu.sync_copy(asrc_hbm.at[c].at[sidx], arows)
        pltpu.sync_copy(adst_hbm.at[c].at[didx], brows)
        for i in range(_CN):
            e16 = arows[i] + brows[i]
            e16 = jnp.maximum(e16, 0.2 * e16)
            x16 = jnp.exp(e16)
            exv[i] = x16
            b0 = _bcast(x16, lane0)
            b1 = _bcast(x16, lane1)
            for d in range(_HID // 16):
                hv = hrows[i, pl.ds(d * 16, 16)]
                scA[i, pl.ds(d * 16, 16)] = hv * b0
                scB[i, pl.ds(d * 16, 16)] = hv * b1
        pltpu.sync_copy(exv, ex_out.at[c, pl.ds(base, _CN)])
        pltpu.sync_copy(exv, den.at[didx], add=True)
        pltpu.sync_copy(scA, accA.at[didx], add=True)
        pltpu.sync_copy(scB, accB.at[didx], add=True)
        return carry

    lax.fori_loop(0, nc, body, 0)
    plsc.subcore_barrier()
    pltpu.sync_copy(accA.at[pl.ds(r0, _RPT)], acc_out.at[c, 0, pl.ds(r0, _RPT)])
    pltpu.sync_copy(accB.at[pl.ds(r0, _RPT)], acc_out.at[c, 1, pl.ds(r0, _RPT)])
    pltpu.sync_copy(den.at[pl.ds(r0, _RPT)], den_out.at[c, pl.ds(r0, _RPT)])


@functools.partial(
    pl.kernel,
    out_type=jax.ShapeDtypeStruct((2, _E, 16), jnp.float32),
    mesh=_mesh,
    compiler_params=_SC_PARAMS,
    scratch_types=[
        pltpu.VMEM((_CN,), jnp.int32),
        pltpu.VMEM((_CN, 16), jnp.float32),
        pltpu.VMEM((_CN, 16), jnp.float32),
        pltpu.VMEM((_CN, 16), jnp.float32),
    ],
)
def _p4_alpha(dst_hbm, ex_hbm, denf_hbm, out_hbm, didx, drows, exv, av):
    c = lax.axis_index("c")
    s = lax.axis_index("s")
    nc = jnp.where(s < _NCHN - _NS * (_NCHN // _NS), _NCHN // _NS + 1,
                   _NCHN // _NS)

    def body(j, carry):
        base = (s + _NS * j) * _CN
        pltpu.sync_copy(dst_hbm.at[pl.ds(base, _CN)], didx)
        pltpu.sync_copy(denf_hbm.at[c].at[didx], drows)
        pltpu.sync_copy(ex_hbm.at[c, pl.ds(base, _CN)], exv)
        for i in range(_CN):
            av[i] = exv[i] / (drows[i] + 1e-16)
        pltpu.sync_copy(av, out_hbm.at[c, pl.ds(base, _CN)])
        return carry

    lax.fori_loop(0, nc, body, 0)


_BR = 1024
_NB = 10


def _tca_body(x_ref, wc_ref, brow_ref, o_ref):
    o_ref[...] = jnp.dot(x_ref[...], wc_ref[...],
                         preferred_element_type=jnp.float32) + brow_ref[...]


def _tca(x, Wc, brow):
    return pl.pallas_call(
        _tca_body,
        grid=(_NB,),
        in_specs=[
            pl.BlockSpec((_BR, _DIN), lambda i: (i, 0)),
            pl.BlockSpec((_DIN, _DIN), lambda i: (0, 0)),
            pl.BlockSpec((1, _DIN), lambda i: (0, 0)),
        ],
        out_specs=pl.BlockSpec((_BR, _DIN), lambda i: (i, 0)),
        out_shape=jax.ShapeDtypeStruct((_N, _DIN), jnp.float32),
    )(x, Wc, brow)


def _dinv_of(p0b):
    cnt = p0b[0, :, 0] + p0b[1, :, 0]
    return lax.rsqrt(cnt + 1.0)


def _tcb_body(p0_ref, h1p_ref, o_ref):
    dinv = _dinv_of(p0_ref[...])
    o_ref[...] = h1p_ref[...] * dinv[:, None]


def _tcb(p0, h1p):
    return pl.pallas_call(
        _tcb_body,
        grid=(_NB,),
        in_specs=[
            pl.BlockSpec((2, _BR, 8), lambda i: (0, i, 0)),
            pl.BlockSpec((_BR, _HID), lambda i: (i, 0)),
        ],
        out_specs=pl.BlockSpec((_BR, _HID), lambda i: (i, 0)),
        out_shape=jax.ShapeDtypeStruct((_N, _HID), jnp.float32),
    )(p0, h1p)


def _tcc_body(agg_ref, h1t_ref, p0_ref, b1_ref, w2_ref, o_ref):
    dinv = _dinv_of(p0_ref[...])
    a = agg_ref[0] + agg_ref[1] + h1t_ref[...]
    h1 = jnp.maximum(a * dinv[:, None] + b1_ref[...], 0.0)
    o_ref[...] = jnp.dot(h1, w2_ref[...],
                         preferred_element_type=jnp.float32) * dinv[:, None]


def _tcc(agg1, h1t, p0, b1row, W2):
    return pl.pallas_call(
        _tcc_body,
        grid=(_NB,),
        in_specs=[
            pl.BlockSpec((2, _BR, _HID), lambda i: (0, i, 0)),
            pl.BlockSpec((_BR, _HID), lambda i: (i, 0)),
            pl.BlockSpec((2, _BR, 8), lambda i: (0, i, 0)),
            pl.BlockSpec((1, _HID), lambda i: (0, 0)),
            pl.BlockSpec((_HID, _HID), lambda i: (0, 0)),
        ],
        out_specs=pl.BlockSpec((_BR, _HID), lambda i: (i, 0)),
        out_shape=jax.ShapeDtypeStruct((_N, _HID), jnp.float32),
    )(agg1, h1t, p0, b1row, W2)


def _tcd_body(agg_ref, h2t_ref, p0_ref, b2_ref, res_ref, wg_ref, att_ref,
              h_ref, asrc_ref, adst_ref, exs_ref):
    dinv = _dinv_of(p0_ref[...])
    a = agg_ref[0] + agg_ref[1] + h2t_ref[...]
    h2 = jnp.maximum(a * dinv[:, None] + b2_ref[...], 0.0)
    h = h2 + res_ref[...]
    h_ref[...] = h
    wgr = wg_ref[...].reshape(_HID, _NH, _HID)
    att = att_ref[...]
    vsrc = jnp.sum(wgr * att[None, 0:4, :], axis=2)
    vdst = jnp.sum(wgr * att[None, 4:8, :], axis=2)
    asr = jnp.dot(h, vsrc, preferred_element_type=jnp.float32)
    ads = jnp.dot(h, vdst, preferred_element_type=jnp.float32)
    es = asr + ads
    exs = jnp.exp(jnp.maximum(es, 0.2 * es))
    exs_ref[...] = exs
    z14 = jnp.zeros((asr.shape[0], 14), jnp.float32)
    asrc_ref[...] = jnp.stack(
        [jnp.concatenate([asr[:, 0:2], z14], axis=1),
         jnp.concatenate([asr[:, 2:4], z14], axis=1)], axis=0)
    adst_ref[...] = jnp.stack(
        [jnp.concatenate([ads[:, 0:2], z14], axis=1),
         jnp.concatenate([ads[:, 2:4], z14], axis=1)], axis=0)


def _tcd(agg2, h2t, p0, b2row, resid, Wg, attc):
    return pl.pallas_call(
        _tcd_body,
        grid=(_NB,),
        in_specs=[
            pl.BlockSpec((2, _BR, _HID), lambda i: (0, i, 0)),
            pl.BlockSpec((_BR, _HID), lambda i: (i, 0)),
            pl.BlockSpec((2, _BR, 8), lambda i: (0, i, 0)),
            pl.BlockSpec((1, _HID), lambda i: (0, 0)),
            pl.BlockSpec((_BR, _HID), lambda i: (i, 0)),
            pl.BlockSpec((_HID, _NH * _HID), lambda i: (0, 0)),
            pl.BlockSpec((8, _HID), lambda i: (0, 0)),
        ],
        out_specs=[
            pl.BlockSpec((_BR, _HID), lambda i: (i, 0)),
            pl.BlockSpec((2, _BR, 16), lambda i: (0, i, 0)),
            pl.BlockSpec((2, _BR, 16), lambda i: (0, i, 0)),
            pl.BlockSpec((_BR, _NH), lambda i: (i, 0)),
        ],
        out_shape=[
            jax.ShapeDtypeStruct((_N, _HID), jnp.float32),
            jax.ShapeDtypeStruct((2, _N, 16), jnp.float32),
            jax.ShapeDtypeStruct((2, _N, 16), jnp.float32),
            jax.ShapeDtypeStruct((_N, _NH), jnp.float32),
        ],
    )(agg2, h2t, p0, b2row, resid, Wg, attc)


def _tce_body(acc_ref, den_ref, h_ref, exs_ref, wg_ref, bg_ref, bat_ref,
              wf_ref, bf_ref,
              asl_ref, denf_ref, logp_ref, sums_ref, cnt_ref):
    i = pl.program_id(0)
    h = h_ref[...]
    exs = exs_ref[...]
    den4 = jnp.concatenate([den_ref[0, :, 0:2], den_ref[1, :, 0:2]], axis=1)
    denf = den4 + exs
    wgr = wg_ref[...].reshape(_HID, _NH, _HID)
    parts = []
    for hh in range(_NH):
        acch = acc_ref[hh // 2, hh % 2] + exs[:, hh:hh + 1] * h
        num = jnp.dot(acch, wgr[:, hh, :], preferred_element_type=jnp.float32)
        parts.append(num / (denf[:, hh:hh + 1] + 1e-16))
    gat = jnp.concatenate(parts, axis=1)
    hf = jnp.maximum(gat + bg_ref[...], 0.0)
    asl_ref[...] = exs / (denf + 1e-16)
    z14 = jnp.zeros((denf.shape[0], 14), jnp.float32)
    denf_ref[...] = jnp.stack(
        [jnp.concatenate([denf[:, 0:2], z14], axis=1),
         jnp.concatenate([denf[:, 2:4], z14], axis=1)], axis=0)

    rows = lax.broadcasted_iota(jnp.int32, (_BR, _G), 0) + i * _BR
    valid = rows < _N
    gid = lax.broadcasted_iota(jnp.int32, (_BR, _G), 1)
    oh = jnp.where((bat_ref[...] == gid) & valid, 1.0, 0.0)

    @pl.when(i == 0)
    def _():
        sums_ref[...] = jnp.zeros_like(sums_ref)
        cnt_ref[...] = jnp.zeros_like(cnt_ref)

    sums_ref[...] += lax.dot_general(oh, hf, (((0,), (0,)), ((), ())),
                                     preferred_element_type=jnp.float32)
    cnt_ref[...] += lax.dot_general(oh, jnp.ones((_BR, 8), jnp.float32),
                                    (((0,), (0,)), ((), ())),
                                    preferred_element_type=jnp.float32)

    @pl.when(i == _NB - 1)
    def _():
        pooled = sums_ref[...] / jnp.maximum(cnt_ref[...][:, 0:1], 1.0)
        logits = jnp.dot(pooled, wf_ref[...],
                         preferred_element_type=jnp.float32) + bf_ref[...]
        m = jnp.max(logits, axis=1, keepdims=True)
        lse = m + jnp.log(jnp.sum(jnp.exp(logits - m), axis=1, keepdims=True))
        logp_ref[...] = logits - lse


def _tce(acc, den, h, exs, Wg, bgrow, batc, Wf, bfrow):
    return pl.pallas_call(
        _tce_body,
        grid=(_NB,),
        in_specs=[
            pl.BlockSpec((2, 2, _BR, _HID), lambda i: (0, 0, i, 0)),
            pl.BlockSpec((2, _BR, 16), lambda i: (0, i, 0)),
            pl.BlockSpec((_BR, _HID), lambda i: (i, 0)),
            pl.BlockSpec((_BR, _NH), lambda i: (i, 0)),
            pl.BlockSpec((_HID, _NH * _HID), lambda i: (0, 0)),
            pl.BlockSpec((1, _NH * _HID), lambda i: (0, 0)),
            pl.BlockSpec((_BR, 1), lambda i: (i, 0)),
            pl.BlockSpec((_NH * _HID, _OUT), lambda i: (0, 0)),
            pl.BlockSpec((1, _OUT), lambda i: (0, 0)),
        ],
        out_specs=[
            pl.BlockSpec((_BR, _NH), lambda i: (i, 0)),
            pl.BlockSpec((2, _BR, 16), lambda i: (0, i, 0)),
            pl.BlockSpec((_G, _OUT), lambda i: (0, 0)),
            pl.BlockSpec((_G, _NH * _HID), lambda i: (0, 0)),
            pl.BlockSpec((_G, 8), lambda i: (0, 0)),
        ],
        out_shape=[
            jax.ShapeDtypeStruct((_N, _NH), jnp.float32),
            jax.ShapeDtypeStruct((2, _N, 16), jnp.float32),
            jax.ShapeDtypeStruct((_G, _OUT), jnp.float32),
            jax.ShapeDtypeStruct((_G, _NH * _HID), jnp.float32),
            jax.ShapeDtypeStruct((_G, 8), jnp.float32),
        ],
    )(acc, den, h, exs, Wg, bgrow, batc, Wf, bfrow)


def kernel(x, edge_index, batch, W1, b1, W2, b2, Wg, att_src, att_dst, bg,
           Wr, br, Wf, bf):
    src = edge_index[0]
    dst = edge_index[1]

    z64 = jnp.zeros((_NP, _HID), jnp.float32)
    z16 = jnp.zeros((_NP, 16), jnp.float32)
    z8 = jnp.zeros((_NP, 8), jnp.float32)
    ones8 = jnp.concatenate(
        [jnp.ones((_CW, 1), jnp.float32), jnp.zeros((_CW, 7), jnp.float32)],
        axis=1)

    Wc = jnp.concatenate([W1, Wr], axis=1)
    brow = jnp.concatenate([jnp.zeros_like(b1), br])[None, :]

    p0 = _p0_deg(dst, ones8, z8)
    hr = _tca(x, Wc, brow)
    h1p = hr[:, :_HID]
    resid = hr[:, _HID:]

    h1t = _tcb(p0, h1p)
    agg1 = _pgcn(h1t, src, dst, z64)
    h2t = _tcc(agg1, h1t, p0, b1[None, :], W2)
    agg2 = _pgcn(h2t, src, dst, z64)
    attc = jnp.concatenate([att_src, att_dst], axis=0)
    h, asrcT, adstT, exs = _tcd(agg2, h2t, p0, b2[None, :], resid, Wg, attc)

    ex, acc, den = _p3_gat(h, asrcT, adstT, src, dst, z64, z16)
    alpha_self, denfT, logp, _sums, _cnt = _tce(
        acc, den, h, exs, Wg, bg[None, :], batch[:, None].astype(jnp.int32),
        Wf, bf[None, :])

    al = _p4_alpha(dst, ex, denfT)
    alpha_edge = jnp.concatenate([al[0, :, 0:2], al[1, :, 0:2]], axis=1)
    alpha = jnp.concatenate([alpha_edge, alpha_self], axis=0)
    return logp, alpha


# speedup vs baseline: 1.0101x; 1.0101x over previous
import functools

import jax
import jax.numpy as jnp
from jax import lax
from jax.experimental import pallas as pl
from jax.experimental.pallas import tpu as pltpu
from jax.experimental.pallas import tpu_sc as plsc

_N = 10000
_E = 320000
_DIN = 128
_HID = 64
_NH = 4
_OUT = 32
_G = 128

_NS = 16
_NP = 10240
_RPT = _NP // _NS
_CW = 128
_CN = 128
_NCHN = _E // _CN
_NCHW = _E // 2 // _CW

_mesh = plsc.VectorSubcoreMesh(core_axis_name="c", subcore_axis_name="s")
_SC_PARAMS = pltpu.CompilerParams(use_tc_tiling_on_sc=False)

_GDN = lax.GatherDimensionNumbers(
    offset_dims=(), collapsed_slice_dims=(0,), start_index_map=(0,))


def _bcast(x16, idx16):
    return lax.gather(x16, idx16[:, None], _GDN, (1,),
                      mode=lax.GatherScatterMode.PROMISE_IN_BOUNDS)


@functools.partial(
    pl.kernel,
    out_type=jax.ShapeDtypeStruct((2, _NP, 8), jnp.float32),
    mesh=_mesh,
    compiler_params=_SC_PARAMS,
    scratch_types=[
        pltpu.VMEM((_CW,), jnp.int32),
        pltpu.VMEM((_CW, 8), jnp.float32),
        pltpu.VMEM_SHARED((_NP, 8), jnp.float32),
    ],
)
def _p0_deg(dst_hbm, ones_hbm, zero8_hbm, out_hbm, didx, ones_v, acc):
    c = lax.axis_index("c")
    s = lax.axis_index("s")
    r0 = s * _RPT
    pltpu.sync_copy(zero8_hbm.at[pl.ds(r0, _RPT)], acc.at[pl.ds(r0, _RPT)])
    pltpu.sync_copy(ones_hbm, ones_v)
    plsc.subcore_barrier()
    nc = jnp.where(s < _NCHW - _NS * (_NCHW // _NS), _NCHW // _NS + 1,
                   _NCHW // _NS)

    def body(j, carry):
        base = (c * _NCHW + s + _NS * j) * _CW
        pltpu.sync_copy(dst_hbm.at[pl.ds(base, _CW)], didx)
        pltpu.sync_copy(ones_v, acc.at[didx], add=True)
        return carry

    lax.fori_loop(0, nc, body, 0)
    plsc.subcore_barrier()
    pltpu.sync_copy(acc.at[pl.ds(r0, _RPT)], out_hbm.at[c, pl.ds(r0, _RPT)])


@functools.partial(
    pl.kernel,
    out_type=jax.ShapeDtypeStruct((2, _NP, _HID), jnp.float32),
    mesh=_mesh,
    compiler_params=_SC_PARAMS,
    scratch_types=[
        pltpu.VMEM((_CW,), jnp.int32),
        pltpu.VMEM((_CW,), jnp.int32),
        pltpu.VMEM((_CW, _HID), jnp.float32),
        pltpu.VMEM_SHARED((_NP, _HID), jnp.float32),
    ],
)
def _pgcn(tab_hbm, src_hbm, dst_hbm, zero_hbm, out_hbm, sidx, didx, rows, acc):
    c = lax.axis_index("c")
    s = lax.axis_index("s")
    r0 = s * _RPT
    pltpu.sync_copy(zero_hbm.at[pl.ds(r0, _RPT)], acc.at[pl.ds(r0, _RPT)])
    plsc.subcore_barrier()
    nc = jnp.where(s < _NCHW - _NS * (_NCHW // _NS), _NCHW // _NS + 1,
                   _NCHW // _NS)

    def body(j, carry):
        base = (c * _NCHW + s + _NS * j) * _CW
        pltpu.sync_copy(src_hbm.at[pl.ds(base, _CW)], sidx)
        pltpu.sync_copy(dst_hbm.at[pl.ds(base, _CW)], didx)
        pltpu.sync_copy(tab_hbm.at[sidx], rows)
        pltpu.sync_copy(rows, acc.at[didx], add=True)
        return carry

    lax.fori_loop(0, nc, body, 0)
    plsc.subcore_barrier()
    pltpu.sync_copy(acc.at[pl.ds(r0, _RPT)], out_hbm.at[c, pl.ds(r0, _RPT)])


@functools.partial(
    pl.kernel,
    out_type=[
        jax.ShapeDtypeStruct((2, _E, 16), jnp.float32),
        jax.ShapeDtypeStruct((2, 2, _NP, _HID), jnp.float32),
        jax.ShapeDtypeStruct((2, _NP, 16), jnp.float32),
    ],
    mesh=_mesh,
    compiler_params=_SC_PARAMS,
    scratch_types=[
        pltpu.VMEM((_CN,), jnp.int32),
        pltpu.VMEM((_CN,), jnp.int32),
        pltpu.VMEM((_CN, _HID), jnp.float32),
        pltpu.VMEM((_CN, 16), jnp.float32),
        pltpu.VMEM((_CN, 16), jnp.float32),
        pltpu.VMEM((_CN, 16), jnp.float32),
        pltpu.VMEM((_CN, _HID), jnp.float32),
        pltpu.VMEM((_CN, _HID), jnp.float32),
        pltpu.VMEM_SHARED((_NP, _HID), jnp.float32),
        pltpu.VMEM_SHARED((_NP, _HID), jnp.float32),
        pltpu.VMEM_SHARED((_NP, 16), jnp.float32),
    ],
)
def _p3_gat(h_hbm, asrc_hbm, adst_hbm, src_hbm, dst_hbm, zero_hbm, zero16_hbm,
            ex_out, acc_out, den_out,
            sidx, didx, hrows, arows, brows, exv, scA, scB, accA, accB, den):
    c = lax.axis_index("c")
    s = lax.axis_index("s")
    r0 = s * _RPT
    pltpu.sync_copy(zero_hbm.at[pl.ds(r0, _RPT)], accA.at[pl.ds(r0, _RPT)])
    pltpu.sync_copy(zero_hbm.at[pl.ds(r0, _RPT)], accB.at[pl.ds(r0, _RPT)])
    pltpu.sync_copy(zero16_hbm.at[pl.ds(r0, _RPT)], den.at[pl.ds(r0, _RPT)])
    plsc.subcore_barrier()
    nc = jnp.where(s < _NCHN - _NS * (_NCHN // _NS), _NCHN // _NS + 1,
                   _NCHN // _NS)
    lane0 = jnp.zeros((16,), jnp.int32)
    lane1 = jnp.ones((16,), jnp.int32)

    def body(j, carry):
        chunk = s + _NS * j
        base = chunk * _CN
        pltpu.sync_copy(src_hbm.at[pl.ds(base, _CN)], sidx)
        pltpu.sync_copy(dst_hbm.at[pl.ds(base, _CN)], didx)
        pltpu.sync_copy(h_hbm.at[sidx], hrows)
        pltpu.sync_copy(asrc_hbm.at[c].at[sidx], arows)
        pltpu.sync_copy(adst_hbm.at[c].at[didx], brows)
        for i in range(_CN):
            e16 = arows[i] + brows[i]
            e16 = jnp.maximum(e16, 0.2 * e16)
            x16 = jnp.exp(e16)
            exv[i] = x16
            b0 = _bcast(x16, lane0)
            b1 = _bcast(x16, lane1)
            for d in range(_HID // 16):
                hv = hrows[i, pl.ds(d * 16, 16)]
                scA[i, pl.ds(d * 16, 16)] = hv * b0
                scB[i, pl.ds(d * 16, 16)] = hv * b1
        pltpu.sync_copy(exv, ex_out.at[c, pl.ds(base, _CN)])
        pltpu.sync_copy(exv, den.at[didx], add=True)
        pltpu.sync_copy(scA, accA.at[didx], add=True)
        pltpu.sync_copy(scB, accB.at[didx], add=True)
        return carry

    lax.fori_loop(0, nc, body, 0)
    plsc.subcore_barrier()
    pltpu.sync_copy(accA.at[pl.ds(r0, _RPT)], acc_out.at[c, 0, pl.ds(r0, _RPT)])
    pltpu.sync_copy(accB.at[pl.ds(r0, _RPT)], acc_out.at[c, 1, pl.ds(r0, _RPT)])
    pltpu.sync_copy(den.at[pl.ds(r0, _RPT)], den_out.at[c, pl.ds(r0, _RPT)])


@functools.partial(
    pl.kernel,
    out_type=jax.ShapeDtypeStruct((2, _E, 16), jnp.float32),
    mesh=_mesh,
    compiler_params=_SC_PARAMS,
    scratch_types=[
        pltpu.VMEM((_CN,), jnp.int32),
        pltpu.VMEM((_CN, 16), jnp.float32),
        pltpu.VMEM((_CN, 16), jnp.float32),
        pltpu.VMEM((_CN, 16), jnp.float32),
    ],
)
def _p4_alpha(dst_hbm, ex_hbm, denf_hbm, out_hbm, didx, drows, exv, av):
    c = lax.axis_index("c")
    s = lax.axis_index("s")
    nc = jnp.where(s < _NCHN - _NS * (_NCHN // _NS), _NCHN // _NS + 1,
                   _NCHN // _NS)

    def body(j, carry):
        base = (s + _NS * j) * _CN
        pltpu.sync_copy(dst_hbm.at[pl.ds(base, _CN)], didx)
        pltpu.sync_copy(denf_hbm.at[c].at[didx], drows)
        pltpu.sync_copy(ex_hbm.at[c, pl.ds(base, _CN)], exv)
        for i in range(_CN):
            av[i] = exv[i] / (drows[i] + 1e-16)
        pltpu.sync_copy(av, out_hbm.at[c, pl.ds(base, _CN)])
        return carry

    lax.fori_loop(0, nc, body, 0)


_BR = 1024
_NB = 10


def _tca_body(x_ref, wc_ref, brow_ref, o_ref):
    o_ref[...] = jnp.dot(x_ref[...], wc_ref[...],
                         preferred_element_type=jnp.float32) + brow_ref[...]


def _tca(x, Wc, brow):
    return pl.pallas_call(
        _tca_body,
        grid=(_NB,),
        in_specs=[
            pl.BlockSpec((_BR, _DIN), lambda i: (i, 0)),
            pl.BlockSpec((_DIN, _DIN), lambda i: (0, 0)),
            pl.BlockSpec((1, _DIN), lambda i: (0, 0)),
        ],
        out_specs=pl.BlockSpec((_BR, _DIN), lambda i: (i, 0)),
        out_shape=jax.ShapeDtypeStruct((_N, _DIN), jnp.float32),
    )(x, Wc, brow)


def _dinv_of(p0b):
    cnt = p0b[0, :, 0] + p0b[1, :, 0]
    return lax.rsqrt(cnt + 1.0)


def _tcb_body(p0_ref, h1p_ref, o_ref):
    dinv = _dinv_of(p0_ref[...])
    o_ref[...] = h1p_ref[...] * dinv[:, None]


def _tcb(p0, h1p):
    return pl.pallas_call(
        _tcb_body,
        grid=(_NB,),
        in_specs=[
            pl.BlockSpec((2, _BR, 8), lambda i: (0, i, 0)),
            pl.BlockSpec((_BR, _HID), lambda i: (i, 0)),
        ],
        out_specs=pl.BlockSpec((_BR, _HID), lambda i: (i, 0)),
        out_shape=jax.ShapeDtypeStruct((_N, _HID), jnp.float32),
    )(p0, h1p)


def _tcc_body(agg_ref, h1t_ref, p0_ref, b1_ref, w2_ref, o_ref):
    dinv = _dinv_of(p0_ref[...])
    a = agg_ref[0] + agg_ref[1] + h1t_ref[...]
    h1 = jnp.maximum(a * dinv[:, None] + b1_ref[...], 0.0)
    o_ref[...] = jnp.dot(h1, w2_ref[...],
                         preferred_element_type=jnp.float32) * dinv[:, None]


def _tcc(agg1, h1t, p0, b1row, W2):
    return pl.pallas_call(
        _tcc_body,
        grid=(_NB,),
        in_specs=[
            pl.BlockSpec((2, _BR, _HID), lambda i: (0, i, 0)),
            pl.BlockSpec((_BR, _HID), lambda i: (i, 0)),
            pl.BlockSpec((2, _BR, 8), lambda i: (0, i, 0)),
            pl.BlockSpec((1, _HID), lambda i: (0, 0)),
            pl.BlockSpec((_HID, _HID), lambda i: (0, 0)),
        ],
        out_specs=pl.BlockSpec((_BR, _HID), lambda i: (i, 0)),
        out_shape=jax.ShapeDtypeStruct((_N, _HID), jnp.float32),
    )(agg1, h1t, p0, b1row, W2)


def _tcd_body(agg_ref, h2t_ref, p0_ref, b2_ref, res_ref, wg_ref, att_ref,
              h_ref, asrc_ref, adst_ref, exs_ref):
    dinv = _dinv_of(p0_ref[...])
    a = agg_ref[0] + agg_ref[1] + h2t_ref[...]
    h2 = jnp.maximum(a * dinv[:, None] + b2_ref[...], 0.0)
    h = h2 + res_ref[...]
    h_ref[...] = h
    wgr = wg_ref[...].reshape(_HID, _NH, _HID)
    att = att_ref[...]
    vsrc = jnp.sum(wgr * att[None, 0:4, :], axis=2)
    vdst = jnp.sum(wgr * att[None, 4:8, :], axis=2)
    asr = jnp.dot(h, vsrc, preferred_element_type=jnp.float32)
    ads = jnp.dot(h, vdst, preferred_element_type=jnp.float32)
    es = asr + ads
    exs = jnp.exp(jnp.maximum(es, 0.2 * es))
    exs_ref[...] = exs
    z14 = jnp.zeros((asr.shape[0], 14), jnp.float32)
    asrc_ref[...] = jnp.stack(
        [jnp.concatenate([asr[:, 0:2], z14], axis=1),
         jnp.concatenate([asr[:, 2:4], z14], axis=1)], axis=0)
    adst_ref[...] = jnp.stack(
        [jnp.concatenate([ads[:, 0:2], z14], axis=1),
         jnp.concatenate([ads[:, 2:4], z14], axis=1)], axis=0)


def _tcd(agg2, h2t, p0, b2row, resid, Wg, attc):
    return pl.pallas_call(
        _tcd_body,
        grid=(_NB,),
        in_specs=[
            pl.BlockSpec((2, _BR, _HID), lambda i: (0, i, 0)),
            pl.BlockSpec((_BR, _HID), lambda i: (i, 0)),
            pl.BlockSpec((2, _BR, 8), lambda i: (0, i, 0)),
            pl.BlockSpec((1, _HID), lambda i: (0, 0)),
            pl.BlockSpec((_BR, _HID), lambda i: (i, 0)),
            pl.BlockSpec((_HID, _NH * _HID), lambda i: (0, 0)),
            pl.BlockSpec((8, _HID), lambda i: (0, 0)),
        ],
        out_specs=[
            pl.BlockSpec((_BR, _HID), lambda i: (i, 0)),
            pl.BlockSpec((2, _BR, 16), lambda i: (0, i, 0)),
            pl.BlockSpec((2, _BR, 16), lambda i: (0, i, 0)),
            pl.BlockSpec((_BR, _NH), lambda i: (i, 0)),
        ],
        out_shape=[
            jax.ShapeDtypeStruct((_N, _HID), jnp.float32),
            jax.ShapeDtypeStruct((2, _N, 16), jnp.float32),
            jax.ShapeDtypeStruct((2, _N, 16), jnp.float32),
            jax.ShapeDtypeStruct((_N, _NH), jnp.float32),
        ],
    )(agg2, h2t, p0, b2row, resid, Wg, attc)


def _tce1_body(den_ref, exs_ref, asl_ref, denf_ref):
    exs = exs_ref[...]
    den4 = jnp.concatenate([den_ref[0, :, 0:2], den_ref[1, :, 0:2]], axis=1)
    denf = den4 + exs
    asl_ref[...] = exs / (denf + 1e-16)
    z14 = jnp.zeros((denf.shape[0], 14), jnp.float32)
    denf_ref[...] = jnp.stack(
        [jnp.concatenate([denf[:, 0:2], z14], axis=1),
         jnp.concatenate([denf[:, 2:4], z14], axis=1)], axis=0)


def _tce1(den, exs):
    return pl.pallas_call(
        _tce1_body,
        grid=(_NB,),
        in_specs=[
            pl.BlockSpec((2, _BR, 16), lambda i: (0, i, 0)),
            pl.BlockSpec((_BR, _NH), lambda i: (i, 0)),
        ],
        out_specs=[
            pl.BlockSpec((_BR, _NH), lambda i: (i, 0)),
            pl.BlockSpec((2, _BR, 16), lambda i: (0, i, 0)),
        ],
        out_shape=[
            jax.ShapeDtypeStruct((_N, _NH), jnp.float32),
            jax.ShapeDtypeStruct((2, _N, 16), jnp.float32),
        ],
    )(den, exs)


def _tce_body(acc_ref, den_ref, h_ref, exs_ref, wg_ref, bg_ref, bat_ref,
              wf_ref, bf_ref, logp_ref, sums_ref, cnt_ref):
    i = pl.program_id(0)
    h = h_ref[...]
    exs = exs_ref[...]
    den4 = jnp.concatenate([den_ref[0, :, 0:2], den_ref[1, :, 0:2]], axis=1)
    denf = den4 + exs
    wgr = wg_ref[...].reshape(_HID, _NH, _HID)
    parts = []
    for hh in range(_NH):
        acch = acc_ref[hh // 2, hh % 2] + exs[:, hh:hh + 1] * h
        num = jnp.dot(acch, wgr[:, hh, :], preferred_element_type=jnp.float32)
        parts.append(num / (denf[:, hh:hh + 1] + 1e-16))
    gat = jnp.concatenate(parts, axis=1)
    hf = jnp.maximum(gat + bg_ref[...], 0.0)

    rows = lax.broadcasted_iota(jnp.int32, (_BR, _G), 0) + i * _BR
    valid = rows < _N
    gid = lax.broadcasted_iota(jnp.int32, (_BR, _G), 1)
    oh = jnp.where((bat_ref[...] == gid) & valid, 1.0, 0.0)

    @pl.when(i == 0)
    def _():
        sums_ref[...] = jnp.zeros_like(sums_ref)
        cnt_ref[...] = jnp.zeros_like(cnt_ref)

    sums_ref[...] += lax.dot_general(oh, hf, (((0,), (0,)), ((), ())),
                                     preferred_element_type=jnp.float32)
    cnt_ref[...] += lax.dot_general(oh, jnp.ones((_BR, 8), jnp.float32),
                                    (((0,), (0,)), ((), ())),
                                    preferred_element_type=jnp.float32)

    @pl.when(i == _NB - 1)
    def _():
        pooled = sums_ref[...] / jnp.maximum(cnt_ref[...][:, 0:1], 1.0)
        logits = jnp.dot(pooled, wf_ref[...],
                         preferred_element_type=jnp.float32) + bf_ref[...]
        m = jnp.max(logits, axis=1, keepdims=True)
        lse = m + jnp.log(jnp.sum(jnp.exp(logits - m), axis=1, keepdims=True))
        logp_ref[...] = logits - lse


def _tce(acc, den, h, exs, Wg, bgrow, batc, Wf, bfrow):
    return pl.pallas_call(
        _tce_body,
        grid=(_NB,),
        in_specs=[
            pl.BlockSpec((2, 2, _BR, _HID), lambda i: (0, 0, i, 0)),
            pl.BlockSpec((2, _BR, 16), lambda i: (0, i, 0)),
            pl.BlockSpec((_BR, _HID), lambda i: (i, 0)),
            pl.BlockSpec((_BR, _NH), lambda i: (i, 0)),
            pl.BlockSpec((_HID, _NH * _HID), lambda i: (0, 0)),
            pl.BlockSpec((1, _NH * _HID), lambda i: (0, 0)),
            pl.BlockSpec((_BR, 1), lambda i: (i, 0)),
            pl.BlockSpec((_NH * _HID, _OUT), lambda i: (0, 0)),
            pl.BlockSpec((1, _OUT), lambda i: (0, 0)),
        ],
        out_specs=[
            pl.BlockSpec((_G, _OUT), lambda i: (0, 0)),
            pl.BlockSpec((_G, _NH * _HID), lambda i: (0, 0)),
            pl.BlockSpec((_G, 8), lambda i: (0, 0)),
        ],
        out_shape=[
            jax.ShapeDtypeStruct((_G, _OUT), jnp.float32),
            jax.ShapeDtypeStruct((_G, _NH * _HID), jnp.float32),
            jax.ShapeDtypeStruct((_G, 8), jnp.float32),
        ],
    )(acc, den, h, exs, Wg, bgrow, batc, Wf, bfrow)


def kernel(x, edge_index, batch, W1, b1, W2, b2, Wg, att_src, att_dst, bg,
           Wr, br, Wf, bf):
    src = edge_index[0]
    dst = edge_index[1]

    z64 = jnp.zeros((_NP, _HID), jnp.float32)
    z16 = jnp.zeros((_NP, 16), jnp.float32)
    z8 = jnp.zeros((_NP, 8), jnp.float32)
    ones8 = jnp.concatenate(
        [jnp.ones((_CW, 1), jnp.float32), jnp.zeros((_CW, 7), jnp.float32)],
        axis=1)

    Wc = jnp.concatenate([W1, Wr], axis=1)
    brow = jnp.concatenate([jnp.zeros_like(b1), br])[None, :]

    p0 = _p0_deg(dst, ones8, z8)
    hr = _tca(x, Wc, brow)
    h1p = hr[:, :_HID]
    resid = hr[:, _HID:]

    h1t = _tcb(p0, h1p)
    agg1 = _pgcn(h1t, src, dst, z64)
    h2t = _tcc(agg1, h1t, p0, b1[None, :], W2)
    agg2 = _pgcn(h2t, src, dst, z64)
    attc = jnp.concatenate([att_src, att_dst], axis=0)
    h, asrcT, adstT, exs = _tcd(agg2, h2t, p0, b2[None, :], resid, Wg, attc)

    ex, acc, den = _p3_gat(h, asrcT, adstT, src, dst, z64, z16)
    alpha_self, denfT = _tce1(den, exs)
    al = _p4_alpha(dst, ex, denfT)
    logp, _sums, _cnt = _tce(
        acc, den, h, exs, Wg, bg[None, :], batch[:, None].astype(jnp.int32),
        Wf, bf[None, :])

    alpha_edge = jnp.concatenate([al[0, :, 0:2], al[1, :, 0:2]], axis=1)
    alpha = jnp.concatenate([alpha_edge, alpha_self], axis=0)
    return logp, alpha


# speedup vs baseline: 1.0160x; 1.0058x over previous
import functools

import jax
import jax.numpy as jnp
from jax import lax
from jax.experimental import pallas as pl
from jax.experimental.pallas import tpu as pltpu
from jax.experimental.pallas import tpu_sc as plsc

_N = 10000
_E = 320000
_DIN = 128
_HID = 64
_NH = 4
_OUT = 32
_G = 128

_NS = 16
_NP = 10240
_RPT = _NP // _NS
_CW = 128
_CN = 128
_NCHN = _E // _CN
_NCHW = _E // 2 // _CW

_mesh = plsc.VectorSubcoreMesh(core_axis_name="c", subcore_axis_name="s")
_SC_PARAMS = pltpu.CompilerParams(use_tc_tiling_on_sc=False)

_GDN = lax.GatherDimensionNumbers(
    offset_dims=(), collapsed_slice_dims=(0,), start_index_map=(0,))


def _bcast(x16, idx16):
    return lax.gather(x16, idx16[:, None], _GDN, (1,),
                      mode=lax.GatherScatterMode.PROMISE_IN_BOUNDS)


@functools.partial(
    pl.kernel,
    out_type=jax.ShapeDtypeStruct((2, _NP, 8), jnp.float32),
    mesh=_mesh,
    compiler_params=_SC_PARAMS,
    scratch_types=[
        pltpu.VMEM((_CW,), jnp.int32),
        pltpu.VMEM((_CW, 8), jnp.float32),
        pltpu.VMEM_SHARED((_NP, 8), jnp.float32),
    ],
)
def _p0_deg(dst_hbm, ones_hbm, zero8_hbm, out_hbm, didx, ones_v, acc):
    c = lax.axis_index("c")
    s = lax.axis_index("s")
    r0 = s * _RPT
    pltpu.sync_copy(zero8_hbm.at[pl.ds(r0, _RPT)], acc.at[pl.ds(r0, _RPT)])
    pltpu.sync_copy(ones_hbm, ones_v)
    plsc.subcore_barrier()
    nc = jnp.where(s < _NCHW - _NS * (_NCHW // _NS), _NCHW // _NS + 1,
                   _NCHW // _NS)

    def body(j, carry):
        base = (c * _NCHW + s + _NS * j) * _CW
        pltpu.sync_copy(dst_hbm.at[pl.ds(base, _CW)], didx)
        pltpu.sync_copy(ones_v, acc.at[didx], add=True)
        return carry

    lax.fori_loop(0, nc, body, 0)
    plsc.subcore_barrier()
    pltpu.sync_copy(acc.at[pl.ds(r0, _RPT)], out_hbm.at[c, pl.ds(r0, _RPT)])


@functools.partial(
    pl.kernel,
    out_type=jax.ShapeDtypeStruct((2, _NP, _HID), jnp.float32),
    mesh=_mesh,
    compiler_params=_SC_PARAMS,
    scratch_types=[
        pltpu.VMEM((_CW,), jnp.int32),
        pltpu.VMEM((_CW,), jnp.int32),
        pltpu.VMEM((_CW, _HID), jnp.float32),
        pltpu.VMEM_SHARED((_NP, _HID), jnp.float32),
    ],
)
def _pgcn(tab_hbm, src_hbm, dst_hbm, zero_hbm, out_hbm, sidx, didx, rows, acc):
    c = lax.axis_index("c")
    s = lax.axis_index("s")
    r0 = s * _RPT
    pltpu.sync_copy(zero_hbm.at[pl.ds(r0, _RPT)], acc.at[pl.ds(r0, _RPT)])
    plsc.subcore_barrier()
    nc = jnp.where(s < _NCHW - _NS * (_NCHW // _NS), _NCHW // _NS + 1,
                   _NCHW // _NS)

    def body(j, carry):
        base = (c * _NCHW + s + _NS * j) * _CW
        pltpu.sync_copy(src_hbm.at[pl.ds(base, _CW)], sidx)
        pltpu.sync_copy(dst_hbm.at[pl.ds(base, _CW)], didx)
        pltpu.sync_copy(tab_hbm.at[sidx], rows)
        pltpu.sync_copy(rows, acc.at[didx], add=True)
        return carry

    lax.fori_loop(0, nc, body, 0)
    plsc.subcore_barrier()
    pltpu.sync_copy(acc.at[pl.ds(r0, _RPT)], out_hbm.at[c, pl.ds(r0, _RPT)])


@functools.partial(
    pl.kernel,
    out_type=[
        jax.ShapeDtypeStruct((2, _E, 16), jnp.float32),
        jax.ShapeDtypeStruct((2, _NP, 2 * _HID), jnp.float32),
        jax.ShapeDtypeStruct((2, _NP, 16), jnp.float32),
    ],
    mesh=_mesh,
    compiler_params=_SC_PARAMS,
    scratch_types=[
        pltpu.VMEM((_CN,), jnp.int32),
        pltpu.VMEM((_CN,), jnp.int32),
        pltpu.VMEM((_CN, _HID), jnp.float32),
        pltpu.VMEM((_CN, 16), jnp.float32),
        pltpu.VMEM((_CN, 16), jnp.float32),
        pltpu.VMEM((_CN, 16), jnp.float32),
        pltpu.VMEM((_CN, 2 * _HID), jnp.float32),
        pltpu.VMEM_SHARED((_NP, 2 * _HID), jnp.float32),
        pltpu.VMEM_SHARED((_NP, 16), jnp.float32),
    ],
)
def _p3_gat(h_hbm, asrc_hbm, adst_hbm, src_hbm, dst_hbm, zero128_hbm, zero16_hbm,
            ex_out, acc_out, den_out,
            sidx, didx, hrows, arows, brows, exv, scAB, accAB, den):
    c = lax.axis_index("c")
    s = lax.axis_index("s")
    r0 = s * _RPT
    pltpu.sync_copy(zero128_hbm.at[pl.ds(r0, _RPT)], accAB.at[pl.ds(r0, _RPT)])
    pltpu.sync_copy(zero16_hbm.at[pl.ds(r0, _RPT)], den.at[pl.ds(r0, _RPT)])
    plsc.subcore_barrier()
    nc = jnp.where(s < _NCHN - _NS * (_NCHN // _NS), _NCHN // _NS + 1,
                   _NCHN // _NS)
    lane0 = jnp.zeros((16,), jnp.int32)
    lane1 = jnp.ones((16,), jnp.int32)

    def body(j, carry):
        chunk = s + _NS * j
        base = chunk * _CN
        pltpu.sync_copy(src_hbm.at[pl.ds(base, _CN)], sidx)
        pltpu.sync_copy(dst_hbm.at[pl.ds(base, _CN)], didx)
        pltpu.sync_copy(h_hbm.at[sidx], hrows)
        pltpu.sync_copy(asrc_hbm.at[c].at[sidx], arows)
        pltpu.sync_copy(adst_hbm.at[c].at[didx], brows)
        for i in range(_CN):
            e16 = arows[i] + brows[i]
            e16 = jnp.maximum(e16, 0.2 * e16)
            x16 = jnp.exp(e16)
            exv[i] = x16
            b0 = _bcast(x16, lane0)
            b1 = _bcast(x16, lane1)
            for d in range(_HID // 16):
                hv = hrows[i, pl.ds(d * 16, 16)]
                scAB[i, pl.ds(d * 16, 16)] = hv * b0
                scAB[i, pl.ds(_HID + d * 16, 16)] = hv * b1
        pltpu.sync_copy(exv, ex_out.at[c, pl.ds(base, _CN)])
        pltpu.sync_copy(exv, den.at[didx], add=True)
        pltpu.sync_copy(scAB, accAB.at[didx], add=True)
        return carry

    lax.fori_loop(0, nc, body, 0)
    plsc.subcore_barrier()
    pltpu.sync_copy(accAB.at[pl.ds(r0, _RPT)], acc_out.at[c, pl.ds(r0, _RPT)])
    pltpu.sync_copy(den.at[pl.ds(r0, _RPT)], den_out.at[c, pl.ds(r0, _RPT)])


@functools.partial(
    pl.kernel,
    out_type=jax.ShapeDtypeStruct((2, _E, 16), jnp.float32),
    mesh=_mesh,
    compiler_params=_SC_PARAMS,
    scratch_types=[
        pltpu.VMEM((_CN,), jnp.int32),
        pltpu.VMEM((_CN, 16), jnp.float32),
        pltpu.VMEM((_CN, 16), jnp.float32),
        pltpu.VMEM((_CN, 16), jnp.float32),
    ],
)
def _p4_alpha(dst_hbm, ex_hbm, denf_hbm, out_hbm, didx, drows, exv, av):
    c = lax.axis_index("c")
    s = lax.axis_index("s")
    nc = jnp.where(s < _NCHN - _NS * (_NCHN // _NS), _NCHN // _NS + 1,
                   _NCHN // _NS)

    def body(j, carry):
        base = (s + _NS * j) * _CN
        pltpu.sync_copy(dst_hbm.at[pl.ds(base, _CN)], didx)
        pltpu.sync_copy(denf_hbm.at[c].at[didx], drows)
        pltpu.sync_copy(ex_hbm.at[c, pl.ds(base, _CN)], exv)
        for i in range(_CN):
            av[i] = exv[i] / (drows[i] + 1e-16)
        pltpu.sync_copy(av, out_hbm.at[c, pl.ds(base, _CN)])
        return carry

    lax.fori_loop(0, nc, body, 0)


_BR = 1024
_NB = 10


def _tca_body(x_ref, wc_ref, brow_ref, o_ref):
    o_ref[...] = jnp.dot(x_ref[...], wc_ref[...],
                         preferred_element_type=jnp.float32) + brow_ref[...]


def _tca(x, Wc, brow):
    return pl.pallas_call(
        _tca_body,
        grid=(_NB,),
        in_specs=[
            pl.BlockSpec((_BR, _DIN), lambda i: (i, 0)),
            pl.BlockSpec((_DIN, _DIN), lambda i: (0, 0)),
            pl.BlockSpec((1, _DIN), lambda i: (0, 0)),
        ],
        out_specs=pl.BlockSpec((_BR, _DIN), lambda i: (i, 0)),
        out_shape=jax.ShapeDtypeStruct((_N, _DIN), jnp.float32),
    )(x, Wc, brow)


def _dinv_of(p0b):
    cnt = p0b[0, :, 0] + p0b[1, :, 0]
    return lax.rsqrt(cnt + 1.0)


def _tcb_body(p0_ref, h1p_ref, o_ref):
    dinv = _dinv_of(p0_ref[...])
    o_ref[...] = h1p_ref[...] * dinv[:, None]


def _tcb(p0, h1p):
    return pl.pallas_call(
        _tcb_body,
        grid=(_NB,),
        in_specs=[
            pl.BlockSpec((2, _BR, 8), lambda i: (0, i, 0)),
            pl.BlockSpec((_BR, _HID), lambda i: (i, 0)),
        ],
        out_specs=pl.BlockSpec((_BR, _HID), lambda i: (i, 0)),
        out_shape=jax.ShapeDtypeStruct((_N, _HID), jnp.float32),
    )(p0, h1p)


def _tcc_body(agg_ref, h1t_ref, p0_ref, b1_ref, w2_ref, o_ref):
    dinv = _dinv_of(p0_ref[...])
    a = agg_ref[0] + agg_ref[1] + h1t_ref[...]
    h1 = jnp.maximum(a * dinv[:, None] + b1_ref[...], 0.0)
    o_ref[...] = jnp.dot(h1, w2_ref[...],
                         preferred_element_type=jnp.float32) * dinv[:, None]


def _tcc(agg1, h1t, p0, b1row, W2):
    return pl.pallas_call(
        _tcc_body,
        grid=(_NB,),
        in_specs=[
            pl.BlockSpec((2, _BR, _HID), lambda i: (0, i, 0)),
            pl.BlockSpec((_BR, _HID), lambda i: (i, 0)),
            pl.BlockSpec((2, _BR, 8), lambda i: (0, i, 0)),
            pl.BlockSpec((1, _HID), lambda i: (0, 0)),
            pl.BlockSpec((_HID, _HID), lambda i: (0, 0)),
        ],
        out_specs=pl.BlockSpec((_BR, _HID), lambda i: (i, 0)),
        out_shape=jax.ShapeDtypeStruct((_N, _HID), jnp.float32),
    )(agg1, h1t, p0, b1row, W2)


def _tcd_body(agg_ref, h2t_ref, p0_ref, b2_ref, res_ref, wg_ref, att_ref,
              h_ref, asrc_ref, adst_ref, exs_ref):
    dinv = _dinv_of(p0_ref[...])
    a = agg_ref[0] + agg_ref[1] + h2t_ref[...]
    h2 = jnp.maximum(a * dinv[:, None] + b2_ref[...], 0.0)
    h = h2 + res_ref[...]
    h_ref[...] = h
    wgr = wg_ref[...].reshape(_HID, _NH, _HID)
    att = att_ref[...]
    vsrc = jnp.sum(wgr * att[None, 0:4, :], axis=2)
    vdst = jnp.sum(wgr * att[None, 4:8, :], axis=2)
    asr = jnp.dot(h, vsrc, preferred_element_type=jnp.float32)
    ads = jnp.dot(h, vdst, preferred_element_type=jnp.float32)
    es = asr + ads
    exs = jnp.exp(jnp.maximum(es, 0.2 * es))
    exs_ref[...] = exs
    z14 = jnp.zeros((asr.shape[0], 14), jnp.float32)
    asrc_ref[...] = jnp.stack(
        [jnp.concatenate([asr[:, 0:2], z14], axis=1),
         jnp.concatenate([asr[:, 2:4], z14], axis=1)], axis=0)
    adst_ref[...] = jnp.stack(
        [jnp.concatenate([ads[:, 0:2], z14], axis=1),
         jnp.concatenate([ads[:, 2:4], z14], axis=1)], axis=0)


def _tcd(agg2, h2t, p0, b2row, resid, Wg, attc):
    return pl.pallas_call(
        _tcd_body,
        grid=(_NB,),
        in_specs=[
            pl.BlockSpec((2, _BR, _HID), lambda i: (0, i, 0)),
            pl.BlockSpec((_BR, _HID), lambda i: (i, 0)),
            pl.BlockSpec((2, _BR, 8), lambda i: (0, i, 0)),
            pl.BlockSpec((1, _HID), lambda i: (0, 0)),
            pl.BlockSpec((_BR, _HID), lambda i: (i, 0)),
            pl.BlockSpec((_HID, _NH * _HID), lambda i: (0, 0)),
            pl.BlockSpec((8, _HID), lambda i: (0, 0)),
        ],
        out_specs=[
            pl.BlockSpec((_BR, _HID), lambda i: (i, 0)),
            pl.BlockSpec((2, _BR, 16), lambda i: (0, i, 0)),
            pl.BlockSpec((2, _BR, 16), lambda i: (0, i, 0)),
            pl.BlockSpec((_BR, _NH), lambda i: (i, 0)),
        ],
        out_shape=[
            jax.ShapeDtypeStruct((_N, _HID), jnp.float32),
            jax.ShapeDtypeStruct((2, _N, 16), jnp.float32),
            jax.ShapeDtypeStruct((2, _N, 16), jnp.float32),
            jax.ShapeDtypeStruct((_N, _NH), jnp.float32),
        ],
    )(agg2, h2t, p0, b2row, resid, Wg, attc)


def _tce1_body(den_ref, exs_ref, asl_ref, denf_ref):
    exs = exs_ref[...]
    den4 = jnp.concatenate([den_ref[0, :, 0:2], den_ref[1, :, 0:2]], axis=1)
    denf = den4 + exs
    asl_ref[...] = exs / (denf + 1e-16)
    z14 = jnp.zeros((denf.shape[0], 14), jnp.float32)
    denf_ref[...] = jnp.stack(
        [jnp.concatenate([denf[:, 0:2], z14], axis=1),
         jnp.concatenate([denf[:, 2:4], z14], axis=1)], axis=0)


def _tce1(den, exs):
    return pl.pallas_call(
        _tce1_body,
        grid=(_NB,),
        in_specs=[
            pl.BlockSpec((2, _BR, 16), lambda i: (0, i, 0)),
            pl.BlockSpec((_BR, _NH), lambda i: (i, 0)),
        ],
        out_specs=[
            pl.BlockSpec((_BR, _NH), lambda i: (i, 0)),
            pl.BlockSpec((2, _BR, 16), lambda i: (0, i, 0)),
        ],
        out_shape=[
            jax.ShapeDtypeStruct((_N, _NH), jnp.float32),
            jax.ShapeDtypeStruct((2, _N, 16), jnp.float32),
        ],
    )(den, exs)


def _tce_body(acc_ref, den_ref, h_ref, exs_ref, wg_ref, bg_ref, bat_ref,
              wf_ref, bf_ref, logp_ref, sums_ref, cnt_ref):
    i = pl.program_id(0)
    h = h_ref[...]
    exs = exs_ref[...]
    den4 = jnp.concatenate([den_ref[0, :, 0:2], den_ref[1, :, 0:2]], axis=1)
    denf = den4 + exs
    wgr = wg_ref[...].reshape(_HID, _NH, _HID)
    parts = []
    for hh in range(_NH):
        acch = acc_ref[hh // 2][:, (hh % 2) * _HID:(hh % 2 + 1) * _HID] \
            + exs[:, hh:hh + 1] * h
        num = jnp.dot(acch, wgr[:, hh, :], preferred_element_type=jnp.float32)
        parts.append(num / (denf[:, hh:hh + 1] + 1e-16))
    gat = jnp.concatenate(parts, axis=1)
    hf = jnp.maximum(gat + bg_ref[...], 0.0)

    rows = lax.broadcasted_iota(jnp.int32, (_BR, _G), 0) + i * _BR
    valid = rows < _N
    gid = lax.broadcasted_iota(jnp.int32, (_BR, _G), 1)
    oh = jnp.where((bat_ref[...] == gid) & valid, 1.0, 0.0)

    @pl.when(i == 0)
    def _():
        sums_ref[...] = jnp.zeros_like(sums_ref)
        cnt_ref[...] = jnp.zeros_like(cnt_ref)

    sums_ref[...] += lax.dot_general(oh, hf, (((0,), (0,)), ((), ())),
                                     preferred_element_type=jnp.float32)
    cnt_ref[...] += lax.dot_general(oh, jnp.ones((_BR, 8), jnp.float32),
                                    (((0,), (0,)), ((), ())),
                                    preferred_element_type=jnp.float32)

    @pl.when(i == _NB - 1)
    def _():
        pooled = sums_ref[...] / jnp.maximum(cnt_ref[...][:, 0:1], 1.0)
        logits = jnp.dot(pooled, wf_ref[...],
                         preferred_element_type=jnp.float32) + bf_ref[...]
        m = jnp.max(logits, axis=1, keepdims=True)
        lse = m + jnp.log(jnp.sum(jnp.exp(logits - m), axis=1, keepdims=True))
        logp_ref[...] = logits - lse


def _tce(acc, den, h, exs, Wg, bgrow, batc, Wf, bfrow):
    return pl.pallas_call(
        _tce_body,
        grid=(_NB,),
        in_specs=[
            pl.BlockSpec((2, _BR, 2 * _HID), lambda i: (0, i, 0)),
            pl.BlockSpec((2, _BR, 16), lambda i: (0, i, 0)),
            pl.BlockSpec((_BR, _HID), lambda i: (i, 0)),
            pl.BlockSpec((_BR, _NH), lambda i: (i, 0)),
            pl.BlockSpec((_HID, _NH * _HID), lambda i: (0, 0)),
            pl.BlockSpec((1, _NH * _HID), lambda i: (0, 0)),
            pl.BlockSpec((_BR, 1), lambda i: (i, 0)),
            pl.BlockSpec((_NH * _HID, _OUT), lambda i: (0, 0)),
            pl.BlockSpec((1, _OUT), lambda i: (0, 0)),
        ],
        out_specs=[
            pl.BlockSpec((_G, _OUT), lambda i: (0, 0)),
            pl.BlockSpec((_G, _NH * _HID), lambda i: (0, 0)),
            pl.BlockSpec((_G, 8), lambda i: (0, 0)),
        ],
        out_shape=[
            jax.ShapeDtypeStruct((_G, _OUT), jnp.float32),
            jax.ShapeDtypeStruct((_G, _NH * _HID), jnp.float32),
            jax.ShapeDtypeStruct((_G, 8), jnp.float32),
        ],
    )(acc, den, h, exs, Wg, bgrow, batc, Wf, bfrow)


def kernel(x, edge_index, batch, W1, b1, W2, b2, Wg, att_src, att_dst, bg,
           Wr, br, Wf, bf):
    src = edge_index[0]
    dst = edge_index[1]

    z64 = jnp.zeros((_NP, _HID), jnp.float32)
    z128 = jnp.zeros((_NP, 2 * _HID), jnp.float32)
    z16 = jnp.zeros((_NP, 16), jnp.float32)
    z8 = jnp.zeros((_NP, 8), jnp.float32)
    ones8 = jnp.concatenate(
        [jnp.ones((_CW, 1), jnp.float32), jnp.zeros((_CW, 7), jnp.float32)],
        axis=1)

    Wc = jnp.concatenate([W1, Wr], axis=1)
    brow = jnp.concatenate([jnp.zeros_like(b1), br])[None, :]

    p0 = _p0_deg(dst, ones8, z8)
    hr = _tca(x, Wc, brow)
    h1p = hr[:, :_HID]
    resid = hr[:, _HID:]

    h1t = _tcb(p0, h1p)
    agg1 = _pgcn(h1t, src, dst, z64)
    h2t = _tcc(agg1, h1t, p0, b1[None, :], W2)
    agg2 = _pgcn(h2t, src, dst, z64)
    attc = jnp.concatenate([att_src, att_dst], axis=0)
    h, asrcT, adstT, exs = _tcd(agg2, h2t, p0, b2[None, :], resid, Wg, attc)

    ex, acc, den = _p3_gat(h, asrcT, adstT, src, dst, z128, z16)
    alpha_self, denfT = _tce1(den, exs)
    al = _p4_alpha(dst, ex, denfT)
    logp, _sums, _cnt = _tce(
        acc, den, h, exs, Wg, bg[None, :], batch[:, None].astype(jnp.int32),
        Wf, bf[None, :])

    alpha_edge = jnp.concatenate([al[0, :, 0:2], al[1, :, 0:2]], axis=1)
    alpha = jnp.concatenate([alpha_edge, alpha_self], axis=0)
    return logp, alpha
